# baseline (speedup 1.0000x reference)
import functools

import jax
import jax.numpy as jnp
import numpy as np
from jax import lax
from jax.experimental import pallas as pl
from jax.experimental.pallas import tpu as pltpu
from jax.experimental.pallas import tpu_sc as plsc

NUM_NODES = 10000
NUM_EDGES = 160000
DIM = 256
_ALL_POSSIBLE = NUM_NODES**2 - NUM_NODES
_NUM_NEG = _ALL_POSSIBLE - NUM_EDGES
_EPS = -np.log(1.0 - NUM_EDGES / _ALL_POSSIBLE)

_NC = 2
_NS = 16
_NW = _NC * _NS
_EPW = 5120
_E_PAD = _NW * _EPW
_CHUNK = 64
_NCHUNK = _EPW // _CHUNK
_GROUPS = _CHUNK // 16
_PK = DIM // 4

_TD_ROWS = _E_PAD // 128
_VALID_ROWS = NUM_EDGES // 128


def _sc_edge_dots(emd_packed, idx_all):
  mesh = plsc.VectorSubcoreMesh(core_axis_name="c", subcore_axis_name="s")

  @functools.partial(
      pl.kernel,
      mesh=mesh,
      out_type=jax.ShapeDtypeStruct((_E_PAD,), jnp.float32),
      compiler_params=pltpu.CompilerParams(
          use_tc_tiling_on_sc=False, needs_layout_passes=False
      ),
      scratch_types=[
          pltpu.VMEM((_NCHUNK, 2 * _CHUNK), jnp.int32),
          pltpu.VMEM((2 * _CHUNK, _PK), jnp.int32),
          pltpu.VMEM((2 * _CHUNK, _PK), jnp.int32),
          pltpu.VMEM((_EPW,), jnp.float32),
          pltpu.VMEM((_PK, 16), jnp.int32),
          pltpu.VMEM_SHARED((NUM_NODES, _PK), jnp.int32),
          pltpu.SemaphoreType.DMA,
          pltpu.SemaphoreType.DMA,
      ],
  )
  def k(emd_hbm, idx_hbm, out_hbm, idx_v, buf0_v, buf1_v, td_v, cols_v,
        table_sh, sem0, sem1):
    wid = lax.axis_index("s") * _NC + lax.axis_index("c")
    base = wid * _EPW
    pltpu.sync_copy(idx_hbm.at[wid], idx_v)
    bufs = (buf0_v, buf1_v)
    sems = (sem0, sem1)

    sid = lax.axis_index("s")
    rows_per_sub = NUM_NODES // _NS
    pltpu.sync_copy(
        emd_hbm.at[pl.ds(sid * rows_per_sub, rows_per_sub)],
        table_sh.at[pl.ds(sid * rows_per_sub, rows_per_sub)],
    )
    plsc.subcore_barrier()

    def start(c, b):
      pltpu.async_copy(table_sh.at[idx_v.at[c]], bufs[b], sems[b])

    def wait(b):
      pltpu.make_async_copy(
          emd_hbm.at[pl.ds(0, 2 * _CHUNK)], bufs[b], sems[b]
      ).wait()

    lanes = lax.iota(jnp.int32, 16)

    for col in range(_PK):
      cols_v[col, pl.ds(0, 16)] = (lanes + col) & (_PK - 1)

    def compute(c, b):
      buf = bufs[b]
      for g in range(_GROUPS):
        rows_a = lanes + (g * 16)
        rows_b = rows_a + _CHUNK
        accs = tuple(jnp.zeros((32,), jnp.bfloat16) for _ in range(8))

        def dim_body(j, accs):
          accs = list(accs)
          for t in range(16):
            col = j * 16 + t
            cols = cols_v[col, pl.ds(0, 16)]
            va = plsc.load_gather(buf, [rows_a, cols])
            vb = plsc.load_gather(buf, [rows_b, cols])
            ae, ao = plsc.unpack(
                plsc.bitcast(va, jnp.float8_e4m3fn),
                format=plsc.PackFormat.INTERLEAVED,
                preferred_element_type=jnp.bfloat16)
            be, bo = plsc.unpack(
                plsc.bitcast(vb, jnp.float8_e4m3fn),
                format=plsc.PackFormat.INTERLEAVED,
                preferred_element_type=jnp.bfloat16)
            accs[t % 8] = accs[t % 8] + (ae * be + ao * bo)
          return tuple(accs)

        accs = lax.fori_loop(0, _PK // 16, dim_body, accs)
        fs = []
        for a0, a1 in (accs[:2], accs[2:4], accs[4:6], accs[6:]):
          s0, s1 = plsc.unpack(a0 + a1, format=plsc.PackFormat.INTERLEAVED)
          fs.append(s0 + s1)
        td_v[pl.ds(c * _CHUNK + g * 16, 16)] = (fs[0] + fs[1]) + (fs[2] + fs[3])

    start(0, 0)

    def pair_body(i, carry):
      c0 = i * 2
      start(c0 + 1, 1)
      wait(0)
      compute(c0, 0)
      start(c0 + 2, 0)
      wait(1)
      compute(c0 + 1, 1)
      return carry

    lax.fori_loop(0, _NCHUNK // 2 - 1, pair_body, 0)
    start(_NCHUNK - 1, 1)
    wait(0)
    compute(_NCHUNK - 2, 0)
    wait(1)
    compute(_NCHUNK - 1, 1)

    pltpu.sync_copy(td_v, out_hbm.at[pl.ds(base, _EPW)])

  return k(emd_packed, idx_all)


def _tc_combine_body(emd_ref, td_ref, out_ref):
  e = emd_ref[...]
  colsum = jnp.sum(e, axis=0)
  total_dot = jnp.sum(colsum * colsum)
  ssq = jnp.sum(e * e)
  td = td_ref[...] + jnp.float32(_EPS)
  rowid = lax.broadcasted_iota(jnp.int32, (_TD_ROWS, 128), 0)
  valid = rowid < _VALID_ROWS
  s_sum = jnp.sum(jnp.where(valid, td, 0.0))
  s_log = jnp.sum(jnp.where(valid, jnp.log(1.0 - jnp.exp(-td)), 0.0))
  te_prob = -s_log / jnp.float32(NUM_EDGES)
  ne_prob = (total_dot - ssq - s_sum) / jnp.float32(_NUM_NEG)
  res = (te_prob + ne_prob) * jnp.float32(0.5)
  out_ref[...] = jnp.broadcast_to(res, (1, 1))


def kernel(emd, edge_index):
  te = jnp.pad(edge_index, ((0, 0), (0, _E_PAD - NUM_EDGES)))
  te1w = te[0].reshape(_NW, _NCHUNK, _CHUNK)
  te2w = te[1].reshape(_NW, _NCHUNK, _CHUNK)
  idx_all = jnp.stack([te1w, te2w], axis=2).reshape(_NW, _NCHUNK, 2 * _CHUNK)
  emd_packed = lax.bitcast_convert_type(
      emd.astype(jnp.float8_e4m3fn).reshape(NUM_NODES, _PK, 4), jnp.int32
  )
  tdot = _sc_edge_dots(emd_packed, idx_all)
  out = pl.pallas_call(
      _tc_combine_body,
      out_shape=jax.ShapeDtypeStruct((1, 1), jnp.float32),
      in_specs=[
          pl.BlockSpec(memory_space=pltpu.VMEM),
          pl.BlockSpec(memory_space=pltpu.VMEM),
      ],
      out_specs=pl.BlockSpec(memory_space=pltpu.VMEM),
  )(emd, tdot.reshape(_TD_ROWS, 128))
  return out.reshape(())


# speedup vs baseline: 1.0527x; 1.0527x over previous
import functools

import jax
import jax.numpy as jnp
import numpy as np
from jax import lax
from jax.experimental import pallas as pl
from jax.experimental.pallas import tpu as pltpu
from jax.experimental.pallas import tpu_sc as plsc

NUM_NODES = 10000
NUM_EDGES = 160000
DIM = 256
_ALL_POSSIBLE = NUM_NODES**2 - NUM_NODES
_NUM_NEG = _ALL_POSSIBLE - NUM_EDGES
_EPS = -np.log(1.0 - NUM_EDGES / _ALL_POSSIBLE)

_NC = 2
_NS = 16
_NW = _NC * _NS
_EPW = 5120
_E_PAD = _NW * _EPW
_CHUNK = 64
_NCHUNK = _EPW // _CHUNK
_GROUPS = _CHUNK // 16
_PK = DIM // 4

_TD_ROWS = _E_PAD // 128
_VALID_ROWS = NUM_EDGES // 128


def _sc_edge_dots(emd_packed, idx_all):
  mesh = plsc.VectorSubcoreMesh(core_axis_name="c", subcore_axis_name="s")

  @functools.partial(
      pl.kernel,
      mesh=mesh,
      out_type=jax.ShapeDtypeStruct((_E_PAD,), jnp.float32),
      compiler_params=pltpu.CompilerParams(
          use_tc_tiling_on_sc=False, needs_layout_passes=False
      ),
      scratch_types=[
          pltpu.VMEM((_NCHUNK, 2 * _CHUNK), jnp.int32),
          pltpu.VMEM((2 * _CHUNK, _PK), jnp.int32),
          pltpu.VMEM((2 * _CHUNK, _PK), jnp.int32),
          pltpu.VMEM((_EPW,), jnp.float32),
          pltpu.VMEM((_PK, 16), jnp.int32),
          pltpu.VMEM_SHARED((NUM_NODES, _PK), jnp.int32),
          pltpu.SemaphoreType.DMA,
          pltpu.SemaphoreType.DMA,
      ],
  )
  def k(emd_hbm, idx_hbm, out_hbm, idx_v, buf0_v, buf1_v, td_v, cols_v,
        table_sh, sem0, sem1):
    wid = lax.axis_index("s") * _NC + lax.axis_index("c")
    base = wid * _EPW
    pltpu.sync_copy(idx_hbm.at[wid], idx_v)
    bufs = (buf0_v, buf1_v)
    sems = (sem0, sem1)

    sid = lax.axis_index("s")
    rows_per_sub = NUM_NODES // _NS
    pltpu.sync_copy(
        emd_hbm.at[pl.ds(sid * rows_per_sub, rows_per_sub)],
        table_sh.at[pl.ds(sid * rows_per_sub, rows_per_sub)],
    )
    plsc.subcore_barrier()

    def start(c, b):
      pltpu.async_copy(table_sh.at[idx_v.at[c]], bufs[b], sems[b])

    def wait(b):
      pltpu.make_async_copy(
          emd_hbm.at[pl.ds(0, 2 * _CHUNK)], bufs[b], sems[b]
      ).wait()

    lanes = lax.iota(jnp.int32, 16)

    for col in range(_PK):
      cols_v[col, pl.ds(0, 16)] = (lanes + col) & (_PK - 1)

    def compute(c, b):
      buf = bufs[b]
      for g in range(_GROUPS):
        rows_a = lanes + (g * 16)
        rows_b = rows_a + _CHUNK
        accs = tuple(jnp.zeros((32,), jnp.bfloat16) for _ in range(8))

        def dim_body(j, accs):
          accs = list(accs)
          for t in range(16):
            col = j * 16 + t
            cols = (lanes + col) & (_PK - 1)
            va = plsc.load_gather(buf, [rows_a, cols])
            vb = plsc.load_gather(buf, [rows_b, cols])
            ae, ao = plsc.unpack(
                plsc.bitcast(va, jnp.float8_e4m3fn),
                format=plsc.PackFormat.INTERLEAVED,
                preferred_element_type=jnp.bfloat16)
            be, bo = plsc.unpack(
                plsc.bitcast(vb, jnp.float8_e4m3fn),
                format=plsc.PackFormat.INTERLEAVED,
                preferred_element_type=jnp.bfloat16)
            accs[t % 8] = accs[t % 8] + (ae * be + ao * bo)
          return tuple(accs)

        accs = lax.fori_loop(0, _PK // 16, dim_body, accs)
        fs = []
        for a0, a1 in (accs[:2], accs[2:4], accs[4:6], accs[6:]):
          s0, s1 = plsc.unpack(a0 + a1, format=plsc.PackFormat.INTERLEAVED)
          fs.append(s0 + s1)
        td_v[pl.ds(c * _CHUNK + g * 16, 16)] = (fs[0] + fs[1]) + (fs[2] + fs[3])

    start(0, 0)

    def pair_body(i, carry):
      c0 = i * 2
      start(c0 + 1, 1)
      wait(0)
      compute(c0, 0)
      start(c0 + 2, 0)
      wait(1)
      compute(c0 + 1, 1)
      return carry

    lax.fori_loop(0, _NCHUNK // 2 - 1, pair_body, 0)
    start(_NCHUNK - 1, 1)
    wait(0)
    compute(_NCHUNK - 2, 0)
    wait(1)
    compute(_NCHUNK - 1, 1)

    pltpu.sync_copy(td_v, out_hbm.at[pl.ds(base, _EPW)])

  return k(emd_packed, idx_all)


def _tc_combine_body(emd_ref, td_ref, out_ref):
  e = emd_ref[...]
  colsum = jnp.sum(e, axis=0)
  total_dot = jnp.sum(colsum * colsum)
  ssq = jnp.sum(e * e)
  td = td_ref[...] + jnp.float32(_EPS)
  rowid = lax.broadcasted_iota(jnp.int32, (_TD_ROWS, 128), 0)
  valid = rowid < _VALID_ROWS
  s_sum = jnp.sum(jnp.where(valid, td, 0.0))
  s_log = jnp.sum(jnp.where(valid, jnp.log(1.0 - jnp.exp(-td)), 0.0))
  te_prob = -s_log / jnp.float32(NUM_EDGES)
  ne_prob = (total_dot - ssq - s_sum) / jnp.float32(_NUM_NEG)
  res = (te_prob + ne_prob) * jnp.float32(0.5)
  out_ref[...] = jnp.broadcast_to(res, (1, 1))


def kernel(emd, edge_index):
  te = jnp.pad(edge_index, ((0, 0), (0, _E_PAD - NUM_EDGES)))
  te1w = te[0].reshape(_NW, _NCHUNK, _CHUNK)
  te2w = te[1].reshape(_NW, _NCHUNK, _CHUNK)
  idx_all = jnp.stack([te1w, te2w], axis=2).reshape(_NW, _NCHUNK, 2 * _CHUNK)
  emd_packed = lax.bitcast_convert_type(
      emd.astype(jnp.float8_e4m3fn).reshape(NUM_NODES, _PK, 4), jnp.int32
  )
  tdot = _sc_edge_dots(emd_packed, idx_all)
  out = pl.pallas_call(
      _tc_combine_body,
      out_shape=jax.ShapeDtypeStruct((1, 1), jnp.float32),
      in_specs=[
          pl.BlockSpec(memory_space=pltpu.VMEM),
          pl.BlockSpec(memory_space=pltpu.VMEM),
      ],
      out_specs=pl.BlockSpec(memory_space=pltpu.VMEM),
  )(emd, tdot.reshape(_TD_ROWS, 128))
  return out.reshape(())


# speedup vs baseline: 1.0593x; 1.0063x over previous
import functools

import jax
import jax.numpy as jnp
import numpy as np
from jax import lax
from jax.experimental import pallas as pl
from jax.experimental.pallas import tpu as pltpu
from jax.experimental.pallas import tpu_sc as plsc

NUM_NODES = 10000
NUM_EDGES = 160000
DIM = 256
_ALL_POSSIBLE = NUM_NODES**2 - NUM_NODES
_NUM_NEG = _ALL_POSSIBLE - NUM_EDGES
_EPS = -np.log(1.0 - NUM_EDGES / _ALL_POSSIBLE)

_NC = 2
_NS = 16
_NW = _NC * _NS
_EPW = 5120
_E_PAD = _NW * _EPW
_CHUNK = 64
_NCHUNK = _EPW // _CHUNK
_GROUPS = _CHUNK // 16
_PK = DIM // 4

_TD_ROWS = _E_PAD // 128
_VALID_ROWS = NUM_EDGES // 128


def _sc_edge_dots(emd_packed, idx_all):
  mesh = plsc.VectorSubcoreMesh(core_axis_name="c", subcore_axis_name="s")

  @functools.partial(
      pl.kernel,
      mesh=mesh,
      out_type=jax.ShapeDtypeStruct((_E_PAD,), jnp.float32),
      compiler_params=pltpu.CompilerParams(
          use_tc_tiling_on_sc=False, needs_layout_passes=False
      ),
      scratch_types=[
          pltpu.VMEM((_NCHUNK, 2 * _CHUNK), jnp.int32),
          pltpu.VMEM((2 * _CHUNK, _PK), jnp.int32),
          pltpu.VMEM((2 * _CHUNK, _PK), jnp.int32),
          pltpu.VMEM((_EPW,), jnp.float32),
          pltpu.VMEM_SHARED((NUM_NODES, _PK), jnp.int32),
          pltpu.SemaphoreType.DMA,
          pltpu.SemaphoreType.DMA,
      ],
  )
  def k(emd_hbm, idx_hbm, out_hbm, idx_v, buf0_v, buf1_v, td_v,
        table_sh, sem0, sem1):
    wid = lax.axis_index("s") * _NC + lax.axis_index("c")
    base = wid * _EPW
    pltpu.sync_copy(idx_hbm.at[wid], idx_v)
    bufs = (buf0_v, buf1_v)
    sems = (sem0, sem1)

    sid = lax.axis_index("s")
    rows_per_sub = NUM_NODES // _NS
    pltpu.sync_copy(
        emd_hbm.at[pl.ds(sid * rows_per_sub, rows_per_sub)],
        table_sh.at[pl.ds(sid * rows_per_sub, rows_per_sub)],
    )
    plsc.subcore_barrier()

    def start(c, b):
      pltpu.async_copy(table_sh.at[idx_v.at[c]], bufs[b], sems[b])

    def wait(b):
      pltpu.make_async_copy(
          emd_hbm.at[pl.ds(0, 2 * _CHUNK)], bufs[b], sems[b]
      ).wait()

    lanes = lax.iota(jnp.int32, 16)

    def compute(c, b):
      buf = bufs[b]
      for g in range(_GROUPS):
        rows_a = lanes + (g * 16)
        rows_b = rows_a + _CHUNK
        accs = tuple(jnp.zeros((32,), jnp.bfloat16) for _ in range(4))

        def dim_body(j, accs):
          accs = list(accs)
          for t in range(8):
            col = j * 8 + t
            cols = (lanes + col) & (_PK - 1)
            va = plsc.load_gather(buf, [rows_a, cols])
            vb = plsc.load_gather(buf, [rows_b, cols])
            ae, ao = plsc.unpack(
                plsc.bitcast(va, jnp.float8_e4m3fn),
                format=plsc.PackFormat.INTERLEAVED,
                preferred_element_type=jnp.bfloat16)
            be, bo = plsc.unpack(
                plsc.bitcast(vb, jnp.float8_e4m3fn),
                format=plsc.PackFormat.INTERLEAVED,
                preferred_element_type=jnp.bfloat16)
            accs[t % 4] = accs[t % 4] + (ae * be + ao * bo)
          return tuple(accs)

        accs = lax.fori_loop(0, _PK // 8, dim_body, accs)
        fs = []
        for a in accs:
          a0, a1 = plsc.unpack(a, format=plsc.PackFormat.INTERLEAVED)
          fs.append(a0 + a1)
        td_v[pl.ds(c * _CHUNK + g * 16, 16)] = (fs[0] + fs[1]) + (fs[2] + fs[3])

    start(0, 0)

    def pair_body(i, carry):
      c0 = i * 2
      start(c0 + 1, 1)
      wait(0)
      compute(c0, 0)
      start(c0 + 2, 0)
      wait(1)
      compute(c0 + 1, 1)
      return carry

    lax.fori_loop(0, _NCHUNK // 2 - 1, pair_body, 0)
    start(_NCHUNK - 1, 1)
    wait(0)
    compute(_NCHUNK - 2, 0)
    wait(1)
    compute(_NCHUNK - 1, 1)

    pltpu.sync_copy(td_v, out_hbm.at[pl.ds(base, _EPW)])

  return k(emd_packed, idx_all)


def _tc_combine_body(emd_ref, td_ref, out_ref):
  e = emd_ref[...]
  colsum = jnp.sum(e, axis=0)
  total_dot = jnp.sum(colsum * colsum)
  ssq = jnp.sum(e * e)
  td = td_ref[...] + jnp.float32(_EPS)
  rowid = lax.broadcasted_iota(jnp.int32, (_TD_ROWS, 128), 0)
  valid = rowid < _VALID_ROWS
  s_sum = jnp.sum(jnp.where(valid, td, 0.0))
  s_log = jnp.sum(jnp.where(valid, jnp.log(1.0 - jnp.exp(-td)), 0.0))
  te_prob = -s_log / jnp.float32(NUM_EDGES)
  ne_prob = (total_dot - ssq - s_sum) / jnp.float32(_NUM_NEG)
  res = (te_prob + ne_prob) * jnp.float32(0.5)
  out_ref[...] = jnp.broadcast_to(res, (1, 1))


def kernel(emd, edge_index):
  te = jnp.pad(edge_index, ((0, 0), (0, _E_PAD - NUM_EDGES)))
  te1w = te[0].reshape(_NW, _NCHUNK, _CHUNK)
  te2w = te[1].reshape(_NW, _NCHUNK, _CHUNK)
  idx_all = jnp.stack([te1w, te2w], axis=2).reshape(_NW, _NCHUNK, 2 * _CHUNK)
  emd_packed = lax.bitcast_convert_type(
      emd.astype(jnp.float8_e4m3fn).reshape(NUM_NODES, _PK, 4), jnp.int32
  )
  tdot = _sc_edge_dots(emd_packed, idx_all)
  out = pl.pallas_call(
      _tc_combine_body,
      out_shape=jax.ShapeDtypeStruct((1, 1), jnp.float32),
      in_specs=[
          pl.BlockSpec(memory_space=pltpu.VMEM),
          pl.BlockSpec(memory_space=pltpu.VMEM),
      ],
      out_specs=pl.BlockSpec(memory_space=pltpu.VMEM),
  )(emd, tdot.reshape(_TD_ROWS, 128))
  return out.reshape(())


# speedup vs baseline: 1.0951x; 1.0339x over previous
import functools

import jax
import jax.numpy as jnp
import numpy as np
from jax import lax
from jax.experimental import pallas as pl
from jax.experimental.pallas import tpu as pltpu
from jax.experimental.pallas import tpu_sc as plsc

NUM_NODES = 10000
NUM_EDGES = 160000
DIM = 256
_ALL_POSSIBLE = NUM_NODES**2 - NUM_NODES
_NUM_NEG = _ALL_POSSIBLE - NUM_EDGES
_EPS = -np.log(1.0 - NUM_EDGES / _ALL_POSSIBLE)

_NC = 2
_NS = 16
_NW = _NC * _NS
_EPW = 5120
_E_PAD = _NW * _EPW
_CHUNK = 64
_NCHUNK = _EPW // _CHUNK
_GROUPS = _CHUNK // 16
_PK = DIM // 4

_TD_ROWS = _E_PAD // 128
_VALID_ROWS = NUM_EDGES // 128


def _sc_edge_dots(emd_packed, idx_all):
  mesh = plsc.VectorSubcoreMesh(core_axis_name="c", subcore_axis_name="s")

  @functools.partial(
      pl.kernel,
      mesh=mesh,
      out_type=jax.ShapeDtypeStruct((_E_PAD,), jnp.float32),
      compiler_params=pltpu.CompilerParams(
          use_tc_tiling_on_sc=False, needs_layout_passes=False
      ),
      scratch_types=[
          pltpu.VMEM((_NCHUNK, 2 * _CHUNK), jnp.int32),
          pltpu.VMEM((2 * _CHUNK, _PK), jnp.int32),
          pltpu.VMEM((2 * _CHUNK, _PK), jnp.int32),
          pltpu.VMEM((_EPW,), jnp.float32),
          pltpu.VMEM_SHARED((NUM_NODES, _PK), jnp.int32),
          pltpu.SemaphoreType.DMA,
          pltpu.SemaphoreType.DMA,
      ],
  )
  def k(emd_hbm, idx_hbm, out_hbm, idx_v, buf0_v, buf1_v, td_v,
        table_sh, sem0, sem1):
    wid = lax.axis_index("s") * _NC + lax.axis_index("c")
    base = wid * _EPW
    pltpu.sync_copy(idx_hbm.at[wid], idx_v)
    bufs = (buf0_v, buf1_v)
    sems = (sem0, sem1)

    sid = lax.axis_index("s")
    rows_per_sub = NUM_NODES // _NS
    pltpu.sync_copy(
        emd_hbm.at[pl.ds(sid * rows_per_sub, rows_per_sub)],
        table_sh.at[pl.ds(sid * rows_per_sub, rows_per_sub)],
    )
    plsc.subcore_barrier()

    def start(c, b):
      pltpu.async_copy(table_sh.at[idx_v.at[c]], bufs[b], sems[b])

    def wait(b):
      pltpu.make_async_copy(
          emd_hbm.at[pl.ds(0, 2 * _CHUNK)], bufs[b], sems[b]
      ).wait()

    lanes = lax.iota(jnp.int32, 16)

    def compute(c, b):
      buf = bufs[b]
      for g in range(_GROUPS):
        rows_a = lanes + (g * 16)
        rows_b = rows_a + _CHUNK
        accs = tuple(jnp.zeros((32,), jnp.bfloat16) for _ in range(4))

        def dim_body(j, accs):
          accs = list(accs)
          for t in range(8):
            col = j * 8 + t
            cols = (lanes + col) & (_PK - 1)
            va = plsc.load_gather(buf, [rows_a, cols])
            vb = plsc.load_gather(buf, [rows_b, cols])
            ae, ao = plsc.unpack(
                plsc.bitcast(va, jnp.float8_e4m3fn),
                format=plsc.PackFormat.INTERLEAVED,
                preferred_element_type=jnp.bfloat16)
            be, bo = plsc.unpack(
                plsc.bitcast(vb, jnp.float8_e4m3fn),
                format=plsc.PackFormat.INTERLEAVED,
                preferred_element_type=jnp.bfloat16)
            accs[t % 4] = accs[t % 4] + (ae * be + ao * bo)
          return tuple(accs)

        accs = lax.fori_loop(0, _PK // 8, dim_body, accs)
        fs = []
        for a in accs:
          a0, a1 = plsc.unpack(a, format=plsc.PackFormat.INTERLEAVED)
          fs.append(a0 + a1)
        td_v[pl.ds(c * _CHUNK + g * 16, 16)] = (fs[0] + fs[1]) + (fs[2] + fs[3])

    start(0, 0)

    def pair_body(i, carry):
      c0 = i * 2
      start(c0 + 1, 1)
      wait(0)
      compute(c0, 0)
      start(c0 + 2, 0)
      wait(1)
      compute(c0 + 1, 1)
      return carry

    lax.fori_loop(0, _NCHUNK // 2 - 1, pair_body, 0)
    start(_NCHUNK - 1, 1)
    wait(0)
    compute(_NCHUNK - 2, 0)
    wait(1)
    compute(_NCHUNK - 1, 1)

    pltpu.sync_copy(td_v, out_hbm.at[pl.ds(base, _EPW)])

  return k(emd_packed, idx_all)


def _tc_stats_body(emd_ref, out_ref):
  e = emd_ref[...]
  colsum = jnp.sum(e, axis=0)
  total_dot = jnp.sum(colsum * colsum)
  ssq = jnp.sum(e * e)
  out_ref[...] = jnp.stack([total_dot, ssq]).reshape(1, 2)


def _tc_combine_body(stats_ref, td_ref, out_ref):
  total_dot = stats_ref[0, 0]
  ssq = stats_ref[0, 1]
  td = td_ref[...] + jnp.float32(_EPS)
  rowid = lax.broadcasted_iota(jnp.int32, (_TD_ROWS, 128), 0)
  valid = rowid < _VALID_ROWS
  s_sum = jnp.sum(jnp.where(valid, td, 0.0))
  s_log = jnp.sum(jnp.where(valid, jnp.log(1.0 - jnp.exp(-td)), 0.0))
  te_prob = -s_log / jnp.float32(NUM_EDGES)
  ne_prob = (total_dot - ssq - s_sum) / jnp.float32(_NUM_NEG)
  res = (te_prob + ne_prob) * jnp.float32(0.5)
  out_ref[...] = jnp.broadcast_to(res, (1, 1))


def kernel(emd, edge_index):
  te = jnp.pad(edge_index, ((0, 0), (0, _E_PAD - NUM_EDGES)))
  te1w = te[0].reshape(_NW, _NCHUNK, _CHUNK)
  te2w = te[1].reshape(_NW, _NCHUNK, _CHUNK)
  idx_all = jnp.stack([te1w, te2w], axis=2).reshape(_NW, _NCHUNK, 2 * _CHUNK)
  emd_packed = lax.bitcast_convert_type(
      emd.astype(jnp.float8_e4m3fn).reshape(NUM_NODES, _PK, 4), jnp.int32
  )
  stats = pl.pallas_call(
      _tc_stats_body,
      out_shape=jax.ShapeDtypeStruct((1, 2), jnp.float32),
      in_specs=[pl.BlockSpec(memory_space=pltpu.VMEM)],
      out_specs=pl.BlockSpec(memory_space=pltpu.VMEM),
  )(emd)
  tdot = _sc_edge_dots(emd_packed, idx_all)
  out = pl.pallas_call(
      _tc_combine_body,
      out_shape=jax.ShapeDtypeStruct((1, 1), jnp.float32),
      in_specs=[
          pl.BlockSpec(memory_space=pltpu.SMEM),
          pl.BlockSpec(memory_space=pltpu.VMEM),
      ],
      out_specs=pl.BlockSpec(memory_space=pltpu.VMEM),
  )(stats, tdot.reshape(_TD_ROWS, 128))
  return out.reshape(())
